# Initial kernel scaffold; baseline (speedup 1.0000x reference)
import jax
import jax.numpy as jnp
from jax.experimental import pallas as pl


def kernel(node_embeddings, edge_index, anti_edge_index, theta1, theta2, theta3, num_iterations):
    raise NotImplementedError("write your pallas kernel here")


# speedup vs baseline: 10.4673x; 10.4673x over previous
import functools

import jax
import jax.numpy as jnp
from jax import lax
from jax.experimental import pallas as pl
from jax.experimental.pallas import tpu as pltpu
from jax.experimental.pallas import tpu_sc as plsc

_N = 50000
_NP = 50048
_P = 32
_D = 96

_NT = 16
_CH = 128
_IB = 16
_K = 4
_E_ALIGN = _NT * _IB * _CH
_ACC_ROWS = 51200
_ZB = 64
_ZPT = _ACC_ROWS // _NT // _ZB
_WPT = _NP // _NT


def _mm_body(x1, x2, x3, w, o1, o2, o3):
    x = jnp.concatenate([x1[...], x2[...], x3[...]], axis=1)
    y = jnp.dot(x, w[...], preferred_element_type=jnp.float32)
    o1[...] = y[:, 0:32]
    o2[...] = y[:, 32:64]
    o3[...] = y[:, 64:96]


_MM_BLK = 3128

_mm = pl.pallas_call(
    _mm_body,
    grid=(_NP // _MM_BLK,),
    in_specs=[pl.BlockSpec((_MM_BLK, _P), lambda i: (i, 0))] * 3
    + [pl.BlockSpec((_D, _D), lambda i: (0, 0))],
    out_specs=[pl.BlockSpec((_MM_BLK, _P), lambda i: (i, 0))] * 3,
    out_shape=[jax.ShapeDtypeStruct((_NP, _P), jnp.float32)] * 3,
)

_sc_mesh = plsc.VectorSubcoreMesh(core_axis_name="c", subcore_axis_name="s")


@functools.partial(
    pl.kernel,
    out_type=[jax.ShapeDtypeStruct((_NP, _P), jnp.float32)] * 2,
    mesh=_sc_mesh,
    scratch_types=[
        pltpu.VMEM_SHARED((_ACC_ROWS, _P), jnp.float32),
        pltpu.VMEM((_IB, _CH), jnp.int32),
        pltpu.VMEM((_IB, _CH), jnp.int32),
        pltpu.VMEM((_K, _CH, _P), jnp.float32),
        pltpu.VMEM((_ZB, _P), jnp.float32),
        pltpu.SemaphoreType.DMA,
    ],
    compiler_params=pltpu.CompilerParams(use_tc_tiling_on_sc=False),
)
def _sc_agg(y2, y3, cols_e, rows_e, cols_a, rows_a, agg2, agg3,
            acc, colbuf, rowbuf, gbuf, zbuf, sem):
    c = lax.axis_index("c")
    s = lax.axis_index("s")
    n_blocks = cols_e.shape[0] // (_NT * _IB)

    zero16 = jnp.zeros((16,), jnp.float32)

    def _zrow(i, carry):
        zbuf[i, pl.ds(0, 16)] = zero16
        zbuf[i, pl.ds(16, 16)] = zero16
        return carry

    lax.fori_loop(0, _ZB, _zrow, 0)

    def _zacc(k, carry):
        pltpu.sync_copy(zbuf, acc.at[pl.ds(s * (_ZPT * _ZB) + k * _ZB, _ZB)])
        return carry

    lax.fori_loop(0, _ZPT, _zacc, 0)
    plsc.subcore_barrier()

    def _run(cols, rows, ytab):
        def _blk(b, carry):
            blk0 = (s * n_blocks + b) * _IB
            pltpu.sync_copy(cols.at[pl.ds(blk0, _IB)], colbuf)
            pltpu.sync_copy(rows.at[pl.ds(blk0, _IB)], rowbuf)
            for g in range(_IB // _K):
                descs = [
                    pltpu.async_copy(
                        ytab.at[colbuf.at[g * _K + j]], gbuf.at[j], sem)
                    for j in range(_K)
                ]
                for dsc in descs:
                    dsc.wait()
                for j in range(_K):
                    pltpu.sync_copy(
                        gbuf.at[j], acc.at[rowbuf.at[g * _K + j]], add=True)
            return carry

        lax.fori_loop(0, n_blocks, _blk, 0)

    @pl.when(c == 0)
    def _():
        _run(cols_e, rows_e, y2)

    @pl.when(c == 1)
    def _():
        _run(cols_a, rows_a, y3)

    plsc.subcore_barrier()

    @pl.when(c == 0)
    def _():
        pltpu.sync_copy(acc.at[pl.ds(s * _WPT, _WPT)], agg2.at[pl.ds(s * _WPT, _WPT)])

    @pl.when(c == 1)
    def _():
        pltpu.sync_copy(acc.at[pl.ds(s * _WPT, _WPT)], agg3.at[pl.ds(s * _WPT, _WPT)])


def _prep_edges(edge_index):
    e = edge_index.shape[1]
    e_pad = -(-e // _E_ALIGN) * _E_ALIGN
    pad = e_pad - e
    rows = jnp.concatenate([edge_index[0], jnp.full((pad,), _N, jnp.int32)])
    cols = jnp.concatenate([edge_index[1], jnp.zeros((pad,), jnp.int32)])
    return cols.reshape(e_pad // _CH, _CH), rows.reshape(e_pad // _CH, _CH)


def kernel(node_embeddings, edge_index, anti_edge_index, theta1, theta2, theta3,
           num_iterations=2):
    w = jnp.concatenate([theta1, theta2, theta3], axis=1)
    cols_e, rows_e = _prep_edges(edge_index)
    cols_a, rows_a = _prep_edges(anti_edge_index)

    padded = jnp.pad(node_embeddings, ((0, _NP - _N), (0, 0)))
    x0 = (padded[:, 0:32], padded[:, 32:64], padded[:, 64:96])

    def body(_, xs):
        y1, y2, y3 = _mm(xs[0], xs[1], xs[2], w)
        agg2, agg3 = _sc_agg(y2, y3, cols_e, rows_e, cols_a, rows_a)
        return (y1, agg2, agg3)

    x1, x2, x3 = lax.fori_loop(0, num_iterations, body, x0)
    return jnp.concatenate([x1[:_N], x2[:_N], x3[:_N]], axis=1)


# speedup vs baseline: 11.0112x; 1.0520x over previous
import functools

import jax
import jax.numpy as jnp
from jax import lax
from jax.experimental import pallas as pl
from jax.experimental.pallas import tpu as pltpu
from jax.experimental.pallas import tpu_sc as plsc

_N = 50000
_NP = 50048
_P = 32
_D = 96

_NT = 16
_CH = 128
_IB = 16
_K = 4
_E_ALIGN = _NT * _IB * _CH
_ACC_ROWS = 51200
_ZB = 64
_ZPT = _ACC_ROWS // _NT // _ZB
_WPT = _NP // _NT


def _mm_body(x1, x2, x3, w, o1, o2, o3):
    x = jnp.concatenate([x1[...], x2[...], x3[...]], axis=1)
    y = jnp.dot(x, w[...], preferred_element_type=jnp.float32)
    o1[...] = y[:, 0:32]
    o2[...] = y[:, 32:64]
    o3[...] = y[:, 64:96]


_MM_BLK = 3128

_mm = pl.pallas_call(
    _mm_body,
    grid=(_NP // _MM_BLK,),
    in_specs=[pl.BlockSpec((_MM_BLK, _P), lambda i: (i, 0))] * 3
    + [pl.BlockSpec((_D, _D), lambda i: (0, 0))],
    out_specs=[pl.BlockSpec((_MM_BLK, _P), lambda i: (i, 0))] * 3,
    out_shape=[jax.ShapeDtypeStruct((_NP, _P), jnp.float32)] * 3,
)

_sc_mesh = plsc.VectorSubcoreMesh(core_axis_name="c", subcore_axis_name="s")


@functools.partial(
    pl.kernel,
    out_type=[jax.ShapeDtypeStruct((_NP, _P), jnp.float32)] * 2,
    mesh=_sc_mesh,
    scratch_types=[
        pltpu.VMEM_SHARED((_ACC_ROWS, _P), jnp.float32),
        pltpu.VMEM((_IB, _CH), jnp.int32),
        pltpu.VMEM((_IB, _CH), jnp.int32),
        pltpu.VMEM((_K, _CH, _P), jnp.float32),
        pltpu.VMEM((_ZB, _P), jnp.float32),
        pltpu.SemaphoreType.DMA,
        pltpu.SemaphoreType.DMA,
    ],
    compiler_params=pltpu.CompilerParams(use_tc_tiling_on_sc=False),
)
def _sc_agg(y2, y3, cols_e, rows_e, cols_a, rows_a, agg2, agg3,
            acc, colbuf, rowbuf, gbuf, zbuf, gsem, ssem):
    c = lax.axis_index("c")
    s = lax.axis_index("s")
    n_blocks = cols_e.shape[0] // (_NT * _IB)

    zero16 = jnp.zeros((16,), jnp.float32)

    def _zrow(i, carry):
        zbuf[i, pl.ds(0, 16)] = zero16
        zbuf[i, pl.ds(16, 16)] = zero16
        return carry

    lax.fori_loop(0, _ZB, _zrow, 0)

    def _zacc(k, carry):
        pltpu.sync_copy(zbuf, acc.at[pl.ds(s * (_ZPT * _ZB) + k * _ZB, _ZB)])
        return carry

    lax.fori_loop(0, _ZPT, _zacc, 0)
    plsc.subcore_barrier()

    def _run(cols, rows, ytab):
        n_sets = _IB // 2

        def _blk(b, carry):
            blk0 = (s * n_blocks + b) * _IB
            pltpu.sync_copy(cols.at[pl.ds(blk0, _IB)], colbuf)
            pltpu.sync_copy(rows.at[pl.ds(blk0, _IB)], rowbuf)

            def _gather(ch, slot):
                return pltpu.async_copy(
                    ytab.at[colbuf.at[ch]], gbuf.at[slot], gsem)

            def _scatter(ch, slot):
                return pltpu.async_copy(
                    gbuf.at[slot], acc.at[rowbuf.at[ch]], ssem, add=True)

            gd = [_gather(j, j) for j in range(2)]
            sd = []
            for k in range(n_sets):
                cur = (0, 1) if k % 2 == 0 else (2, 3)
                nxt = (2, 3) if k % 2 == 0 else (0, 1)
                for d in gd:
                    d.wait()
                for d in sd:
                    d.wait()
                if k + 1 < n_sets:
                    gd = [_gather(2 * (k + 1) + j, nxt[j]) for j in range(2)]
                sd = [_scatter(2 * k + j, cur[j]) for j in range(2)]
            for d in sd:
                d.wait()
            return carry

        lax.fori_loop(0, n_blocks, _blk, 0)

    @pl.when(c == 0)
    def _():
        _run(cols_e, rows_e, y2)

    @pl.when(c == 1)
    def _():
        _run(cols_a, rows_a, y3)

    plsc.subcore_barrier()

    @pl.when(c == 0)
    def _():
        pltpu.sync_copy(acc.at[pl.ds(s * _WPT, _WPT)], agg2.at[pl.ds(s * _WPT, _WPT)])

    @pl.when(c == 1)
    def _():
        pltpu.sync_copy(acc.at[pl.ds(s * _WPT, _WPT)], agg3.at[pl.ds(s * _WPT, _WPT)])


def _prep_edges(edge_index):
    e = edge_index.shape[1]
    e_pad = -(-e // _E_ALIGN) * _E_ALIGN
    pad = e_pad - e
    rows = jnp.concatenate([edge_index[0], jnp.full((pad,), _N, jnp.int32)])
    cols = jnp.concatenate([edge_index[1], jnp.zeros((pad,), jnp.int32)])
    return cols.reshape(e_pad // _CH, _CH), rows.reshape(e_pad // _CH, _CH)


def kernel(node_embeddings, edge_index, anti_edge_index, theta1, theta2, theta3,
           num_iterations=2):
    w = jnp.concatenate([theta1, theta2, theta3], axis=1)
    cols_e, rows_e = _prep_edges(edge_index)
    cols_a, rows_a = _prep_edges(anti_edge_index)

    padded = jnp.pad(node_embeddings, ((0, _NP - _N), (0, 0)))
    x0 = (padded[:, 0:32], padded[:, 32:64], padded[:, 64:96])

    def body(_, xs):
        y1, y2, y3 = _mm(xs[0], xs[1], xs[2], w)
        agg2, agg3 = _sc_agg(y2, y3, cols_e, rows_e, cols_a, rows_a)
        return (y1, agg2, agg3)

    x1, x2, x3 = lax.fori_loop(0, num_iterations, body, x0)
    return jnp.concatenate([x1[:_N], x2[:_N], x3[:_N]], axis=1)
